# baseline (speedup 1.0000x reference)
import functools

import jax
import jax.numpy as jnp
from jax import lax
from jax.experimental import pallas as pl
from jax.experimental.pallas import tpu as pltpu
from jax.experimental.pallas import tpu_sc as plsc

VOCAB = 1000000
EMBED = 16
NUM_CLASSES = 2
BATCH = 16384
TOTAL_IDX = BATCH * 2
SR = VOCAB // 8

_info = plsc.get_sparse_core_info()
_NC, _NS = _info.num_cores, _info.num_subcores
_NW = _NC * _NS
_PER_W = TOTAL_IDX // _NW
_NCHUNK = 4
_CH = _PER_W // _NCHUNK
_NG = _CH // 16

_mesh = plsc.VectorSubcoreMesh(core_axis_name="c", subcore_axis_name="s")


@functools.partial(
    pl.kernel,
    mesh=_mesh,
    out_type=jax.ShapeDtypeStruct((TOTAL_IDX,), jnp.float32),
    scratch_types=[
        pltpu.VMEM((_PER_W,), jnp.int32),
        pltpu.VMEM((_CH,), jnp.int32),
        pltpu.VMEM((_CH,), jnp.int32),
        pltpu.VMEM((_CH,), jnp.int32),
        pltpu.VMEM((_CH,), jnp.int32),
        pltpu.VMEM((_CH, 128), jnp.float32),
        pltpu.VMEM((_CH, 128), jnp.float32),
        pltpu.VMEM((_PER_W,), jnp.float32),
        pltpu.VMEM((2 * 16 * 16,), jnp.float32),
        pltpu.VMEM((16,), jnp.float32),
        pltpu.VMEM((32,), jnp.float32),
        pltpu.SemaphoreType.DMA,
        pltpu.SemaphoreType.DMA,
    ],
    compiler_params=pltpu.CompilerParams(needs_layout_passes=False),
)
def _sc_fused(table2_hbm, idx_hbm, w_hbm, bias_hbm, out_hbm,
              idx_v, big0, big1, big2, big3, rows_a, rows_b,
              out_v, w_v, bias_v, tmp_v, sem_a, sem_b):
    wid = lax.axis_index("s") * _NC + lax.axis_index("c")
    base = wid * _PER_W

    pltpu.sync_copy(idx_hbm.at[pl.ds(base, _PER_W)], idx_v)
    pltpu.sync_copy(w_hbm, w_v)
    pltpu.sync_copy(bias_hbm, bias_v)

    bigs = (big0, big1, big2, big3)
    for c in range(_NCHUNK):
        def _big_body(i, _, c=c):
            v = idx_v[pl.ds(c * _CH + i * 16, 16)]
            bigs[c][pl.ds(i * 16, 16)] = lax.shift_right_logical(v, 3)
            return 0
        lax.fori_loop(0, _CH // 16, _big_body, 0)

    bufs = (rows_a, rows_b)
    sems = (sem_a, sem_b)
    copies = [None] * _NCHUNK

    def _issue(c):
        copies[c] = pltpu.async_copy(
            table2_hbm.at[bigs[c]], bufs[c % 2], sems[c % 2])

    _issue(0)
    _issue(1)

    lane = lax.iota(jnp.int32, 16)
    even = (lane & 1) == 0
    perm_a = lane ^ 1
    perm_b = perm_a + 16
    bias = bias_v[pl.ds(0, 16)]

    for c in range(_NCHUNK):
        copies[c].wait()
        if c + 2 < _NCHUNK:
            _issue(c + 2)
        rows = bufs[c % 2]

        def _group_body(g, _, c=c, rows=rows):
            j0 = c * _CH + g * 16
            iv = idx_v[pl.ds(j0, 16)]
            off = (iv & 7) * 16
            rowid = g * 16 + lane
            acc0 = jnp.zeros((16,), jnp.float32)
            acc1 = jnp.zeros((16,), jnp.float32)
            for k in range(EMBED):
                col = off + ((lane + k) & 15)
                val = plsc.load_gather(rows, [rowid, col])
                acc0 = acc0 + val * w_v[pl.ds(k * 16, 16)]
                acc1 = acc1 + val * w_v[pl.ds(256 + k * 16, 16)]
            tmp_v[pl.ds(0, 16)] = acc0
            tmp_v[pl.ds(16, 16)] = acc1
            q0 = plsc.load_gather(tmp_v, [perm_a])
            q1 = plsc.load_gather(tmp_v, [perm_b])
            out_v[pl.ds(j0, 16)] = (
                jnp.where(even, acc0 + q0, acc1 + q1) + bias)
            return 0

        lax.fori_loop(0, _NG, _group_body, 0)

    pltpu.sync_copy(out_v, out_hbm.at[pl.ds(base, _PER_W)])


@jax.jit
def kernel(x, embedding, fc_w, fc_b):
    xf = x.reshape(-1).astype(jnp.int32)
    table2 = embedding.reshape(SR, 128)
    w = fc_w.astype(jnp.float32)
    lane = jnp.arange(16)
    k = jnp.arange(16)[:, None]
    col = (lane & 1) * 16 + ((k + lane) & 15)
    w_rot = jnp.concatenate(
        [w[0][col].reshape(-1), w[1][col].reshape(-1)])
    bias_v = jnp.tile(fc_b.astype(jnp.float32), 8)
    out = _sc_fused(table2, xf, w_rot, bias_v)
    return out.reshape(BATCH, NUM_CLASSES)


# speedup vs baseline: 1.5946x; 1.5946x over previous
import functools

import jax
import jax.numpy as jnp
from jax import lax
from jax.experimental import pallas as pl
from jax.experimental.pallas import tpu as pltpu
from jax.experimental.pallas import tpu_sc as plsc

VOCAB = 1000000
EMBED = 16
NUM_CLASSES = 2
BATCH = 16384
TOTAL_IDX = BATCH * 2
SR = VOCAB // 8

_info = plsc.get_sparse_core_info()
_NC, _NS = _info.num_cores, _info.num_subcores
_NW = _NC * _NS
_PER_W = TOTAL_IDX // _NW
_NCHUNK = 4
_CH = _PER_W // _NCHUNK
_NG = _CH // 16

_mesh = plsc.VectorSubcoreMesh(core_axis_name="c", subcore_axis_name="s")


@functools.partial(
    pl.kernel,
    mesh=_mesh,
    out_type=jax.ShapeDtypeStruct((TOTAL_IDX,), jnp.float32),
    scratch_types=[
        pltpu.VMEM((_PER_W,), jnp.int32),
        pltpu.VMEM((_CH,), jnp.int32),
        pltpu.VMEM((_CH,), jnp.int32),
        pltpu.VMEM((_CH,), jnp.int32),
        pltpu.VMEM((_CH,), jnp.int32),
        pltpu.VMEM((_CH, 128), jnp.float32),
        pltpu.VMEM((_CH, 128), jnp.float32),
        pltpu.VMEM((_PER_W,), jnp.float32),
        pltpu.VMEM((2 * 16 * 16,), jnp.float32),
        pltpu.VMEM((16,), jnp.float32),
        pltpu.VMEM((32,), jnp.float32),
        pltpu.SemaphoreType.DMA,
        pltpu.SemaphoreType.DMA,
    ],
    compiler_params=pltpu.CompilerParams(needs_layout_passes=False),
)
def _sc_fused(table2_hbm, idx_hbm, w_hbm, bias_hbm, out_hbm,
              idx_v, big0, big1, big2, big3, rows_a, rows_b,
              out_v, w_v, bias_v, tmp_v, sem_a, sem_b):
    wid = lax.axis_index("s") * _NC + lax.axis_index("c")
    base = wid * _PER_W

    pltpu.sync_copy(idx_hbm.at[pl.ds(base, _PER_W)], idx_v)
    pltpu.sync_copy(w_hbm, w_v)
    pltpu.sync_copy(bias_hbm, bias_v)

    bigs = (big0, big1, big2, big3)
    for c in range(_NCHUNK):
        def _big_body(i, _, c=c):
            v = idx_v[pl.ds(c * _CH + i * 16, 16)]
            bigs[c][pl.ds(i * 16, 16)] = lax.shift_right_logical(v, 3)
            return 0
        lax.fori_loop(0, _CH // 16, _big_body, 0)

    bufs = (rows_a, rows_b)
    sems = (sem_a, sem_b)
    copies = [None] * _NCHUNK

    def _issue(c):
        copies[c] = pltpu.async_copy(
            table2_hbm.at[bigs[c]], bufs[c % 2], sems[c % 2])

    _issue(0)
    _issue(1)

    lane = lax.iota(jnp.int32, 16)
    even = (lane & 1) == 0
    perm_a = lane ^ 1
    perm_b = perm_a + 16
    bias = bias_v[pl.ds(0, 16)]

    for c in range(_NCHUNK):
        copies[c].wait()
        if c + 2 < _NCHUNK:
            _issue(c + 2)
        rows = bufs[c % 2]

        def _group_body(g, _, c=c, rows=rows):
            j0 = c * _CH + g * 16
            iv = idx_v[pl.ds(j0, 16)]
            off = (iv & 7) * 16
            rowid = g * 16 + lane
            acc0 = jnp.zeros((16,), jnp.float32)
            acc1 = jnp.zeros((16,), jnp.float32)
            for k in range(EMBED):
                col = off + ((lane + k) & 15)
                val = plsc.load_gather(rows, [rowid, col])
                acc0 = acc0 + val * w_v[pl.ds(k * 16, 16)]
                acc1 = acc1 + val * w_v[pl.ds(256 + k * 16, 16)]
            tmp_v[pl.ds(0, 16)] = acc0
            tmp_v[pl.ds(16, 16)] = acc1
            q0 = plsc.load_gather(tmp_v, [perm_a])
            q1 = plsc.load_gather(tmp_v, [perm_b])
            out_v[pl.ds(j0, 16)] = (
                jnp.where(even, acc0 + q0, acc1 + q1) + bias)
            return 0

        lax.fori_loop(0, _NG, _group_body, 0)

    pltpu.sync_copy(out_v, out_hbm.at[pl.ds(base, _PER_W)])


@jax.jit
def kernel(x, embedding, fc_w, fc_b):
    xf = x.reshape(-1).astype(jnp.int32)
    table2 = jnp.transpose(
        embedding.T.reshape(EMBED, SR, 8), (1, 2, 0)).reshape(SR, 128)
    w = fc_w.astype(jnp.float32)
    lane = jnp.arange(16)
    k = jnp.arange(16)[:, None]
    col = (lane & 1) * 16 + ((k + lane) & 15)
    w_rot = jnp.concatenate(
        [w[0][col].reshape(-1), w[1][col].reshape(-1)])
    bias_v = jnp.tile(fc_b.astype(jnp.float32), 8)
    out = _sc_fused(table2, xf, w_rot, bias_v)
    return out.reshape(BATCH, NUM_CLASSES)
